# baseline (speedup 1.0000x reference)
import numpy as np

import jax
import jax.numpy as jnp
from jax.experimental import pallas as pl
from jax.experimental.pallas import tpu as pltpu

N = 10000
NFEAT = 128
NHID = 64
NCLASS = 16
NW = NHID + NCLASS

ROWS = 200
NBI = N // ROWS
RW2 = 800
CW2 = 2048
NG = 13
NBC2 = 5
EDGE = N - (NBC2 - 1) * CW2
SGRP = 3
SROWS = SGRP * RW2


def _cmin_group(g):
    return (RW2 * g) // CW2


_COPY_AT = []
for _m in range(NBC2 - 1):
    _COPY_AT.append(min(
        i for i in range(NBI) if _cmin_group(i // (RW2 // ROWS)) == _m + 1))


def _main_kernel(ia_ref, gb_ref, cb_ref, ph_ref, gg_ref, cc_ref, og_ref,
                 x_ref, adja_ref, adjb_ref, w1_ref, w2_ref, b1_ref, b2_ref,
                 o_ref, s1s2_ref, s2v_ref, part_ref, stash_ref):
    t = pl.program_id(0)
    ph = ph_ref[t]
    g = gg_ref[t]
    c = cc_ref[t]

    @pl.when(t == 0)
    def _():
        s1s2_ref[...] = jnp.zeros_like(s1s2_ref)
        s1s2_ref[0:N, :NHID] = jnp.dot(
            x_ref[...], w1_ref[...],
            preferred_element_type=jnp.float32).astype(jnp.bfloat16)
        s2v_ref[N:, :] = jnp.zeros_like(s2v_ref[N:, :])

    for _m, _strip in enumerate(_COPY_AT):
        @pl.when(jnp.logical_and(ph == 0, g == _strip))
        def _():
            s1s2_ref[_m * CW2:(_m + 1) * CW2, NHID:] = \
                s2v_ref[_m * CW2:(_m + 1) * CW2, :].astype(jnp.bfloat16)

    @pl.when(ph == 0)
    def _():
        abf = adja_ref[...].astype(jnp.bfloat16)
        hp = jnp.dot(abf, s1s2_ref[0:N, :],
                     preferred_element_type=jnp.float32)
        h = jnp.maximum(hp[:, :NHID] + b1_ref[...], 0.0)
        s2_i = jnp.dot(h, w2_ref[...], preferred_element_type=jnp.float32)
        s2v_ref[pl.ds(g * ROWS, ROWS), :] = s2_i
        part_ref[pl.ds(g * ROWS, ROWS), :] = hp[:, NHID:]

        @pl.when(g < SROWS // ROWS)
        def _():
            stash_ref[pl.ds(g * ROWS, ROWS), :] = abf[:, (NBC2 - 1) * CW2:N]

    first = c == (RW2 * g) // CW2

    def _accum_and_finish(contrib, last):
        base = jnp.where(first, part_ref[pl.ds(g * RW2, RW2), :],
                         o_ref[...])
        if not last:
            o_ref[...] = base + contrib
        else:
            z = base + contrib + b2_ref[...]
            m = jnp.max(z, axis=1, keepdims=True)
            shifted = z - m
            lse = jnp.log(jnp.sum(jnp.exp(shifted), axis=1, keepdims=True))
            o_ref[...] = shifted - lse

    @pl.when(jnp.logical_and(ph == 1, c != NBC2 - 1))
    def _():
        contrib = jnp.dot(adjb_ref[...],
                          s2v_ref[pl.ds(c * CW2, CW2), :],
                          preferred_element_type=jnp.float32)
        _accum_and_finish(contrib, last=False)

    edge = jnp.logical_and(ph == 1, c == NBC2 - 1)

    @pl.when(jnp.logical_and(edge, g < SGRP))
    def _():
        ablk = stash_ref[pl.ds(g * RW2, RW2), :]
        s2c = s2v_ref[(NBC2 - 1) * CW2:N, :].astype(jnp.bfloat16)
        contrib = jnp.dot(ablk, s2c, preferred_element_type=jnp.float32)
        _accum_and_finish(contrib, last=True)

    @pl.when(jnp.logical_and(edge, g >= SGRP))
    def _():
        col_ids = jax.lax.broadcasted_iota(jnp.int32, (RW2, CW2), 1)
        blk = jnp.where(col_ids < EDGE, adjb_ref[...], 0.0)
        contrib = jnp.dot(blk,
                          s2v_ref[(NBC2 - 1) * CW2:(NBC2 - 1) * CW2 + CW2, :],
                          preferred_element_type=jnp.float32)
        _accum_and_finish(contrib, last=True)


def _schedule():
    ia, gb, cb, ph, gg, cc, og = [], [], [], [], [], [], []
    for i in range(NBI):
        ia.append(i); gb.append(0); cb.append(0)
        ph.append(0); gg.append(i); cc.append(0); og.append(0)
    last_b = (0, 0)
    for g in range(NG):
        for c in range(_cmin_group(g), NBC2):
            if c != NBC2 - 1 or g >= SGRP:
                last_b = (g, c)
            ia.append(NBI - 1)
            gb.append(last_b[0]); cb.append(last_b[1])
            ph.append(1); gg.append(g); cc.append(c); og.append(g)
    mk = lambda v: np.asarray(v, dtype=np.int32)
    return tuple(mk(v) for v in (ia, gb, cb, ph, gg, cc, og))


_IA, _GB, _CB, _PH, _GG, _CC, _OG = _schedule()
_T = len(_IA)


@jax.jit
def kernel(x, adj, W1, b1, W2, b2):
    b1r = b1.reshape(1, NHID)
    b2r = b2.reshape(1, NCLASS)

    out = pl.pallas_call(
        _main_kernel,
        grid_spec=pltpu.PrefetchScalarGridSpec(
            num_scalar_prefetch=7,
            grid=(_T,),
            in_specs=[
                pl.BlockSpec((N, NFEAT), lambda t, *s: (0, 0)),
                pl.BlockSpec((ROWS, N), lambda t, *s: (s[0][t], 0)),
                pl.BlockSpec((RW2, CW2), lambda t, *s: (s[1][t], s[2][t])),
                pl.BlockSpec((NFEAT, NHID), lambda t, *s: (0, 0)),
                pl.BlockSpec((NHID, NCLASS), lambda t, *s: (0, 0)),
                pl.BlockSpec((1, NHID), lambda t, *s: (0, 0)),
                pl.BlockSpec((1, NCLASS), lambda t, *s: (0, 0)),
            ],
            out_specs=pl.BlockSpec(
                (RW2, NCLASS), lambda t, *s: (s[6][t], 0)),
            scratch_shapes=[
                pltpu.VMEM((N, NW), jnp.bfloat16),
                pltpu.VMEM((NG * RW2 + 32, NCLASS), jnp.float32),
                pltpu.VMEM((NG * RW2 + 32, NCLASS), jnp.float32),
                pltpu.VMEM((SROWS, EDGE), jnp.bfloat16),
            ],
        ),
        out_shape=jax.ShapeDtypeStruct((N, NCLASS), jnp.float32),
        compiler_params=pltpu.CompilerParams(
            dimension_semantics=("arbitrary",),
        ),
    )(jnp.asarray(_IA), jnp.asarray(_GB), jnp.asarray(_CB),
      jnp.asarray(_PH), jnp.asarray(_GG), jnp.asarray(_CC),
      jnp.asarray(_OG), x, adj, adj, W1, W2, b1r, b2r)

    return out
